# baseline (speedup 1.0000x reference)
import functools

import jax
import jax.numpy as jnp
from jax import lax
from jax.experimental import pallas as pl
from jax.experimental.pallas import tpu as pltpu
from jax.experimental.pallas import tpu_sc as plsc

_PRIME = 5099
_E = 8
_TM = 256
_MAX_TILES = 15
_GROWS = 4096
_IB = 896
_NW = 32


def _nt(a, b):
    return lax.dot_general(a, b, (((1,), (1,)), ((), ())),
                           preferred_element_type=jnp.float32)


def _mix_body(hid_ref, sh_ref, tmk_ref, tmr_ref, hk_ref, rin_ref):
    hid = hid_ref[...]
    delta = sh_ref[...] - hid
    hk_ref[...] = (hid + delta * tmk_ref[...]).astype(jnp.bfloat16)
    rin_ref[...] = (hid + delta * tmr_ref[...]).astype(jnp.bfloat16)


def _shared_body(hk_ref, rin_ref, wr_ref, wks_ref, wvs_ref,
                 r_ref, s_ref, wrb_ref, wkb_ref, wvb_ref):
    m = pl.program_id(0)
    k = pl.program_id(1)

    @pl.when(jnp.logical_and(m == 0, k == 0))
    def _():
        wrb_ref[...] = wr_ref[...].astype(jnp.bfloat16)

    @pl.when(m == 0)
    def _():
        wkb_ref[k] = wks_ref[...].astype(jnp.bfloat16)
        wvb_ref[k] = wvs_ref[...].astype(jnp.bfloat16)

    @pl.when(k == 0)
    def _():
        r_ref[...] = jax.nn.sigmoid(
            _nt(rin_ref[...], wrb_ref[...])).astype(jnp.bfloat16)

    h = jnp.maximum(_nt(hk_ref[...], wkb_ref[k]), 0.0)
    h = (h * h).astype(jnp.bfloat16)
    contrib = _nt(h, wvb_ref[k])

    @pl.when(k == 0)
    def _():
        s_ref[...] = contrib

    @pl.when(k > 0)
    def _():
        s_ref[...] += contrib


def _first_tile(te_ref, p):
    prev = te_ref[jnp.maximum(p - 1, 0)]
    return jnp.logical_or(p == 0, te_ref[p] != prev)


def _grouped_body(te_ref, x_ref, wk_ref, wv_ref, o_ref, wkb_ref, wvb_ref):
    p = pl.program_id(0)
    k = pl.program_id(1)

    @pl.when(_first_tile(te_ref, p))
    def _():
        wkb_ref[k] = wk_ref[0].astype(jnp.bfloat16)
        wvb_ref[k] = wv_ref[0].astype(jnp.bfloat16)

    h = jnp.maximum(_nt(x_ref[...], wkb_ref[k]), 0.0)
    h = (h * h).astype(jnp.bfloat16)
    contrib = _nt(h, wvb_ref[k])

    @pl.when(k == 0)
    def _():
        o_ref[...] = contrib

    @pl.when(k > 0)
    def _():
        o_ref[...] += contrib


def _combine_body(r_ref, s_ref, eo_ref, o_ref):
    o_ref[...] = r_ref[...].astype(jnp.float32) * (s_ref[...] + eo_ref[...])


def _sc_gather_rows(table, idx, n_rows, d, dtype):
    bpw = n_rows // _NW
    mesh = plsc.VectorSubcoreMesh(core_axis_name="c", subcore_axis_name="s")

    @functools.partial(
        pl.kernel, mesh=mesh,
        out_type=jax.ShapeDtypeStruct((n_rows, d), dtype),
        scratch_types=[
            pltpu.VMEM((bpw,), jnp.int32),
            pltpu.VMEM((bpw, d), dtype),
            pltpu.SemaphoreType.DMA,
        ],
    )
    def k(table_hbm, idx_hbm, out_hbm, idx_v, rows_v, sem):
        wid = lax.axis_index("s") * 2 + lax.axis_index("c")
        base = wid * bpw
        pltpu.sync_copy(idx_hbm.at[pl.ds(base, bpw)], idx_v)
        pltpu.async_copy(table_hbm.at[idx_v], rows_v, sem).wait()
        pltpu.sync_copy(rows_v, out_hbm.at[pl.ds(base, bpw)])

    return k(table, idx)


def kernel(hidden, input_ids, time_maa_k, time_maa_r, W_r,
           Wk_shared, Wv_shared, Wk_experts, Wv_experts):
    B, S, H = hidden.shape
    I = Wk_shared.shape[0]
    K = I // _IB
    M = S // _TM
    P = _MAX_TILES

    x = hidden.reshape(S, H)
    shifted = jnp.pad(x, ((1, 0), (0, 0)))[:-1, :]
    tmk = time_maa_k.reshape(1, H)
    tmr = time_maa_r.reshape(1, H)

    ids = input_ids.reshape(-1).astype(jnp.int32)
    e_t = (ids * _PRIME) % _E
    onehot = (e_t[:, None] == jnp.arange(_E, dtype=jnp.int32)[None, :])
    ranks = jnp.cumsum(onehot.astype(jnp.int32), axis=0)
    counts = ranks[-1]
    rank_t = jnp.take_along_axis(ranks, e_t[:, None], axis=1)[:, 0] - 1
    tiles_per_e = (counts + _TM - 1) // _TM
    tile_end = jnp.cumsum(tiles_per_e)
    tile_start = tile_end - tiles_per_e
    dest = tile_start[e_t] * _TM + rank_t
    perm = (jnp.arange(_GROWS, dtype=jnp.int32) % S).at[dest].set(
        jnp.arange(S, dtype=jnp.int32))
    tile_expert = jnp.minimum(
        jnp.searchsorted(tile_end, jnp.arange(P), side="right"),
        _E - 1).astype(jnp.int32)

    hk, rin = pl.pallas_call(
        _mix_body,
        grid=(M,),
        in_specs=[
            pl.BlockSpec((_TM, H), lambda m: (m, 0)),
            pl.BlockSpec((_TM, H), lambda m: (m, 0)),
            pl.BlockSpec((1, H), lambda m: (0, 0)),
            pl.BlockSpec((1, H), lambda m: (0, 0)),
        ],
        out_specs=[
            pl.BlockSpec((_TM, H), lambda m: (m, 0)),
            pl.BlockSpec((_TM, H), lambda m: (m, 0)),
        ],
        out_shape=[jax.ShapeDtypeStruct((S, H), jnp.bfloat16)] * 2,
    )(x, shifted, tmk, tmr)

    hk_pk = lax.bitcast_convert_type(hk.reshape(S, H // 2, 2), jnp.int32)
    xs_pk = _sc_gather_rows(hk_pk, perm, _GROWS, H // 2, jnp.int32)
    x_sorted = lax.bitcast_convert_type(xs_pk, jnp.bfloat16).reshape(_GROWS, H)

    recept, shared = pl.pallas_call(
        _shared_body,
        grid=(M, K),
        in_specs=[
            pl.BlockSpec((_TM, H), lambda m, k: (m, 0)),
            pl.BlockSpec((_TM, H), lambda m, k: (m, 0)),
            pl.BlockSpec((H, H), lambda m, k: (0, 0)),
            pl.BlockSpec((_IB, H),
                         lambda m, k: (jnp.where(m == 0, k, K - 1), 0)),
            pl.BlockSpec((H, _IB),
                         lambda m, k: (0, jnp.where(m == 0, k, K - 1))),
        ],
        out_specs=[
            pl.BlockSpec((_TM, H), lambda m, k: (m, 0)),
            pl.BlockSpec((_TM, H), lambda m, k: (m, 0)),
        ],
        out_shape=[jax.ShapeDtypeStruct((S, H), jnp.bfloat16),
                   jax.ShapeDtypeStruct((S, H), jnp.float32)],
        scratch_shapes=[
            pltpu.VMEM((H, H), jnp.bfloat16),
            pltpu.VMEM((K, _IB, H), jnp.bfloat16),
            pltpu.VMEM((K, H, _IB), jnp.bfloat16),
        ],
    )(hk, rin, W_r, Wk_shared, Wv_shared)

    def _x_map(p, k, te):
        return (p, 0)

    def _kk(p, k, te):
        prev = te[jnp.maximum(p - 1, 0)]
        first = jnp.logical_or(p == 0, te[p] != prev)
        return jnp.where(first, k, K - 1)

    def _wk_map(p, k, te):
        return (te[p], _kk(p, k, te), 0)

    def _wv_map(p, k, te):
        return (te[p], 0, _kk(p, k, te))

    out_sorted = pl.pallas_call(
        _grouped_body,
        grid_spec=pltpu.PrefetchScalarGridSpec(
            num_scalar_prefetch=1,
            grid=(P, K),
            in_specs=[
                pl.BlockSpec((_TM, H), _x_map),
                pl.BlockSpec((1, _IB, H), _wk_map),
                pl.BlockSpec((1, H, _IB), _wv_map),
            ],
            out_specs=pl.BlockSpec((_TM, H), _x_map),
            scratch_shapes=[
                pltpu.VMEM((K, _IB, H), jnp.bfloat16),
                pltpu.VMEM((K, H, _IB), jnp.bfloat16),
            ],
        ),
        out_shape=jax.ShapeDtypeStruct((P * _TM, H), jnp.float32),
    )(tile_expert, x_sorted, Wk_experts, Wv_experts)

    expert_out = _sc_gather_rows(out_sorted, dest, S, H, jnp.float32)

    out = pl.pallas_call(
        _combine_body,
        grid=(M,),
        in_specs=[
            pl.BlockSpec((_TM, H), lambda m: (m, 0)),
            pl.BlockSpec((_TM, H), lambda m: (m, 0)),
            pl.BlockSpec((_TM, H), lambda m: (m, 0)),
        ],
        out_specs=pl.BlockSpec((_TM, H), lambda m: (m, 0)),
        out_shape=jax.ShapeDtypeStruct((S, H), jnp.float32),
    )(recept, shared, expert_out)

    return out.reshape(B, S, H)


# speedup vs baseline: 1.3802x; 1.3802x over previous
import functools

import jax
import jax.numpy as jnp
from jax import lax
from jax.experimental import pallas as pl
from jax.experimental.pallas import tpu as pltpu
from jax.experimental.pallas import tpu_sc as plsc

_PRIME = 5099
_E = 8
_TM = 256
_MAX_TILES = 15
_GROWS = 4096
_IB = 896
_NW = 32


def _nt(a, b):
    return lax.dot_general(a, b, (((1,), (1,)), ((), ())),
                           preferred_element_type=jnp.float32)


def _mix_body(hid_ref, sh_ref, tmk_ref, tmr_ref, hk_ref, rin_ref):
    hid = hid_ref[...]
    delta = sh_ref[...] - hid
    hk_ref[...] = hid + delta * tmk_ref[...]
    rin_ref[...] = (hid + delta * tmr_ref[...]).astype(jnp.bfloat16)


def _make_fused_body(M, K):
    def body(te_ref, hk_ref, rin_ref, wr_ref, wks_ref, wvs_ref,
             x_ref, wk_ref, wv_ref,
             r_ref, s_ref, o_ref,
             wrb_ref, wkb_ref, wvb_ref, wkeb_ref, wveb_ref):
        i = pl.program_id(0)
        k = pl.program_id(1)

        @pl.when(i < M)
        def _():
            @pl.when(jnp.logical_and(i == 0, k == 0))
            def _():
                wrb_ref[...] = wr_ref[...].astype(jnp.bfloat16)

            @pl.when(i == 0)
            def _():
                wkb_ref[k] = wks_ref[...].astype(jnp.bfloat16)
                wvb_ref[k] = wvs_ref[...].astype(jnp.bfloat16)

            @pl.when(k == 0)
            def _():
                r_ref[...] = jax.nn.sigmoid(
                    _nt(rin_ref[...], wrb_ref[...])).astype(jnp.bfloat16)

            h = jnp.maximum(
                _nt(hk_ref[...].astype(jnp.bfloat16), wkb_ref[k]), 0.0)
            h = (h * h).astype(jnp.bfloat16)
            contrib = _nt(h, wvb_ref[k])

            @pl.when(k == 0)
            def _():
                s_ref[...] = contrib

            @pl.when(k > 0)
            def _():
                s_ref[...] += contrib

        @pl.when(i >= M)
        def _():
            pp = i - M
            prev = te_ref[jnp.maximum(pp - 1, 0)]
            first = jnp.logical_or(pp == 0, te_ref[pp] != prev)

            @pl.when(first)
            def _():
                wkeb_ref[k] = wk_ref[0].astype(jnp.bfloat16)
                wveb_ref[k] = wv_ref[0].astype(jnp.bfloat16)

            h = jnp.maximum(
                _nt(x_ref[...].astype(jnp.bfloat16), wkeb_ref[k]), 0.0)
            h = (h * h).astype(jnp.bfloat16)
            contrib = _nt(h, wveb_ref[k])

            @pl.when(k == 0)
            def _():
                o_ref[...] = contrib

            @pl.when(k > 0)
            def _():
                o_ref[...] += contrib

    return body


def _combine_body(r_ref, s_ref, eo_ref, o_ref):
    o_ref[...] = r_ref[...].astype(jnp.float32) * (s_ref[...] + eo_ref[...])


def _sc_gather_rows(table, idx, n_rows, d):
    bpw = n_rows // _NW
    mesh = plsc.VectorSubcoreMesh(core_axis_name="c", subcore_axis_name="s")

    @functools.partial(
        pl.kernel, mesh=mesh,
        out_type=jax.ShapeDtypeStruct((n_rows, d), jnp.float32),
        scratch_types=[
            pltpu.VMEM((bpw,), jnp.int32),
            pltpu.VMEM((bpw, d), jnp.float32),
            pltpu.SemaphoreType.DMA,
        ],
    )
    def k(table_hbm, idx_hbm, out_hbm, idx_v, rows_v, sem):
        wid = lax.axis_index("s") * 2 + lax.axis_index("c")
        base = wid * bpw
        pltpu.sync_copy(idx_hbm.at[pl.ds(base, bpw)], idx_v)
        pltpu.async_copy(table_hbm.at[idx_v], rows_v, sem).wait()
        pltpu.sync_copy(rows_v, out_hbm.at[pl.ds(base, bpw)])

    return k(table, idx)


def kernel(hidden, input_ids, time_maa_k, time_maa_r, W_r,
           Wk_shared, Wv_shared, Wk_experts, Wv_experts):
    B, S, H = hidden.shape
    I = Wk_shared.shape[0]
    K = I // _IB
    M = S // _TM
    P = _MAX_TILES

    x = hidden.reshape(S, H)
    shifted = jnp.pad(x, ((1, 0), (0, 0)))[:-1, :]
    tmk = time_maa_k.reshape(1, H)
    tmr = time_maa_r.reshape(1, H)

    ids = input_ids.reshape(-1).astype(jnp.int32)
    e_t = (ids * _PRIME) % _E
    onehot = (e_t[:, None] == jnp.arange(_E, dtype=jnp.int32)[None, :])
    ranks = jnp.cumsum(onehot.astype(jnp.int32), axis=0)
    counts = ranks[-1]
    rank_t = jnp.take_along_axis(ranks, e_t[:, None], axis=1)[:, 0] - 1
    tiles_per_e = (counts + _TM - 1) // _TM
    tile_end = jnp.cumsum(tiles_per_e)
    tile_start = tile_end - tiles_per_e
    dest = tile_start[e_t] * _TM + rank_t
    perm = (jnp.arange(_GROWS, dtype=jnp.int32) % S).at[dest].set(
        jnp.arange(S, dtype=jnp.int32))
    tile_expert = jnp.minimum(
        jnp.searchsorted(tile_end, jnp.arange(P), side="right"),
        _E - 1).astype(jnp.int32)

    hk, rin = pl.pallas_call(
        _mix_body,
        grid=(M,),
        in_specs=[
            pl.BlockSpec((_TM, H), lambda m: (m, 0)),
            pl.BlockSpec((_TM, H), lambda m: (m, 0)),
            pl.BlockSpec((1, H), lambda m: (0, 0)),
            pl.BlockSpec((1, H), lambda m: (0, 0)),
        ],
        out_specs=[
            pl.BlockSpec((_TM, H), lambda m: (m, 0)),
            pl.BlockSpec((_TM, H), lambda m: (m, 0)),
        ],
        out_shape=[jax.ShapeDtypeStruct((S, H), jnp.float32),
                   jax.ShapeDtypeStruct((S, H), jnp.bfloat16)],
    )(x, shifted, tmk, tmr)

    x_sorted = _sc_gather_rows(hk, perm, _GROWS, H)

    def _row_map(i, k, te):
        return (jnp.minimum(i, M - 1), 0)

    def _const_map(i, k, te):
        return (0, 0)

    def _wks_map(i, k, te):
        return (jnp.where(i == 0, k, K - 1), 0)

    def _wvs_map(i, k, te):
        return (0, jnp.where(i == 0, k, K - 1))

    def _pp(i):
        return jnp.clip(i - M, 0, P - 1)

    def _x_map(i, k, te):
        return (_pp(i), 0)

    def _kk_e(i, k, te):
        pp = _pp(i)
        prev = te[jnp.maximum(pp - 1, 0)]
        first = jnp.logical_or(pp == 0, te[pp] != prev)
        return jnp.where(jnp.logical_and(i >= M, first), k, K - 1)

    def _wke_map(i, k, te):
        return (te[_pp(i)], _kk_e(i, k, te), 0)

    def _wve_map(i, k, te):
        return (te[_pp(i)], 0, _kk_e(i, k, te))

    recept, shared, out_sorted = pl.pallas_call(
        _make_fused_body(M, K),
        grid_spec=pltpu.PrefetchScalarGridSpec(
            num_scalar_prefetch=1,
            grid=(M + P, K),
            in_specs=[
                pl.BlockSpec((_TM, H), _row_map),
                pl.BlockSpec((_TM, H), _row_map),
                pl.BlockSpec((H, H), _const_map),
                pl.BlockSpec((_IB, H), _wks_map),
                pl.BlockSpec((H, _IB), _wvs_map),
                pl.BlockSpec((_TM, H), _x_map),
                pl.BlockSpec((1, _IB, H), _wke_map),
                pl.BlockSpec((1, H, _IB), _wve_map),
            ],
            out_specs=[
                pl.BlockSpec((_TM, H), _row_map),
                pl.BlockSpec((_TM, H), _row_map),
                pl.BlockSpec((_TM, H), _x_map),
            ],
            scratch_shapes=[
                pltpu.VMEM((H, H), jnp.bfloat16),
                pltpu.VMEM((K, _IB, H), jnp.bfloat16),
                pltpu.VMEM((K, H, _IB), jnp.bfloat16),
                pltpu.VMEM((K, _IB, H), jnp.bfloat16),
                pltpu.VMEM((K, H, _IB), jnp.bfloat16),
            ],
        ),
        out_shape=[jax.ShapeDtypeStruct((S, H), jnp.bfloat16),
                   jax.ShapeDtypeStruct((S, H), jnp.float32),
                   jax.ShapeDtypeStruct((P * _TM, H), jnp.float32)],
    )(tile_expert, hk, rin, W_r, Wk_shared, Wv_shared,
      x_sorted, Wk_experts, Wv_experts)

    expert_out = _sc_gather_rows(out_sorted, dest, S, H)

    out = pl.pallas_call(
        _combine_body,
        grid=(M,),
        in_specs=[
            pl.BlockSpec((_TM, H), lambda m: (m, 0)),
            pl.BlockSpec((_TM, H), lambda m: (m, 0)),
            pl.BlockSpec((_TM, H), lambda m: (m, 0)),
        ],
        out_specs=pl.BlockSpec((_TM, H), lambda m: (m, 0)),
        out_shape=jax.ShapeDtypeStruct((S, H), jnp.float32),
    )(recept, shared, expert_out)

    return out.reshape(B, S, H)
